# baseline (speedup 1.0000x reference)
import functools

import jax
import jax.numpy as jnp
from jax.experimental import pallas as pl
from jax.experimental.pallas import tpu as pltpu

N = 4096
IN_F = 256
OUT_F = 64
BM = 128
NB = N // BM
DEPTH = 8


def _l2norm_sigmoid(x):
    n = jnp.sqrt(jnp.sum(x * x, axis=1, keepdims=True))
    return jax.nn.sigmoid(x / jnp.maximum(n, 1e-12))


def _stream_copy(j, adj_ref, gn_ref, sbuf_ref, sem_ref):
    slot = jax.lax.rem(j, DEPTH)
    adj_cp = pltpu.make_async_copy(
        adj_ref.at[pl.ds(j * BM, BM), :], sbuf_ref.at[slot], sem_ref.at[slot])
    gn_cp = pltpu.make_async_copy(
        gn_ref.at[pl.ds((j - NB) * BM, BM), :], sbuf_ref.at[slot],
        sem_ref.at[slot])
    return adj_cp, gn_cp


def _k_main(feat_ref, feata_ref, w1_ref, adj_ref, gn_ref, w2_ref, dw_ref,
            db_ref, hid_ref, h_ref, ret_ref, reta_ref, m1_ref, zc_ref,
            abf_ref, sbuf_ref, sem_ref):
    i = pl.program_id(0)

    @pl.when(i == 0)
    def _prologue():
        for j in range(DEPTH):
            adj_cp, _ = _stream_copy(jnp.int32(j), adj_ref, gn_ref,
                                     sbuf_ref, sem_ref)
            adj_cp.start()
        w1 = w1_ref[...]
        m1_ref[:, :OUT_F] = jnp.dot(feat_ref[...], w1,
                                    preferred_element_type=jnp.float32)
        m1_ref[:, OUT_F:] = jnp.dot(feata_ref[...], w1,
                                    preferred_element_type=jnp.float32)

    slot = jax.lax.rem(i, DEPTH)
    wait_adj, wait_gn = _stream_copy(i, adj_ref, gn_ref, sbuf_ref, sem_ref)

    @pl.when(i < NB)
    def _wait_b():
        wait_adj.wait()

    @pl.when(i >= NB)
    def _wait_c():
        wait_gn.wait()

    @pl.when(i < NB)
    def _encode():
        ablk = sbuf_ref[slot]
        zb = jnp.dot(ablk, m1_ref[...], preferred_element_type=jnp.float32)
        zc_ref[pl.ds(i * BM, BM), :] = zb
        hid_ref[...] = zb[:, :OUT_F]
        abf_ref[pl.ds(i * BM, BM), :] = ablk.astype(jnp.bfloat16)

    @pl.when(i >= NB)
    def _block():
        r = i - NB
        zcat = zc_ref[...]
        gn = sbuf_ref[slot]

        abf = abf_ref[pl.ds(r * BM, BM), :]
        y = jnp.dot(abf, zcat[:, :OUT_F].astype(jnp.bfloat16),
                    preferred_element_type=jnp.float32)
        h_ref[...] = jnp.dot(y, w2_ref[...], preferred_element_type=jnp.float32)

        emb_full = jnp.maximum(zcat, 0.0)
        vsum = jnp.dot(gn, emb_full, preferred_element_type=jnp.float32)
        rowsum = jnp.sum(gn, axis=1, keepdims=True)
        g = _l2norm_sigmoid(vsum[:, :OUT_F] / rowsum)
        ga = _l2norm_sigmoid(vsum[:, OUT_F:] / rowsum)

        zblk = zc_ref[pl.ds(r * BM, BM), :]
        emb = jnp.maximum(zblk[:, :OUT_F], 0.0)
        emba = jnp.maximum(zblk[:, OUT_F:], 0.0)
        dw = dw_ref[...]
        t = jnp.dot(emb, dw, preferred_element_type=jnp.float32)
        ta = jnp.dot(emba, dw, preferred_element_type=jnp.float32)
        b = db_ref[0, 0]
        sc1 = jnp.sum(t * g, axis=1, keepdims=True) + b
        sc2 = jnp.sum(ta * g, axis=1, keepdims=True) + b
        ret_ref[...] = jnp.concatenate([sc1, sc2], axis=1)
        sc1a = jnp.sum(ta * ga, axis=1, keepdims=True) + b
        sc2a = jnp.sum(t * ga, axis=1, keepdims=True) + b
        reta_ref[...] = jnp.concatenate([sc1a, sc2a], axis=1)

    nxt = i + DEPTH
    next_adj, next_gn = _stream_copy(nxt, adj_ref, gn_ref, sbuf_ref, sem_ref)

    @pl.when(nxt < NB)
    def _issue_b():
        next_adj.start()

    @pl.when(jnp.logical_and(nxt >= NB, nxt < 2 * NB))
    def _issue_c():
        next_gn.start()


@functools.partial(jax.jit, static_argnames=("interpret",))
def kernel(feat, feat_a, adj, graph_neigh, weight1, weight2, disc_w, disc_b,
           interpret=False):
    f32 = jnp.float32
    const = lambda i: (0, 0)
    adj_map = lambda i: (jnp.minimum(i, NB - 1), 0)
    gn_map = lambda i: (jnp.maximum(i - NB, 0), 0)
    hiden_emb, h, ret, ret_a = pl.pallas_call(
        _k_main,
        grid=(2 * NB,),
        in_specs=[
            pl.BlockSpec((N, IN_F), const),
            pl.BlockSpec((N, IN_F), const),
            pl.BlockSpec((IN_F, OUT_F), const),
            pl.BlockSpec(memory_space=pltpu.MemorySpace.HBM),
            pl.BlockSpec(memory_space=pltpu.MemorySpace.HBM),
            pl.BlockSpec((OUT_F, IN_F), const),
            pl.BlockSpec((OUT_F, OUT_F), const),
            pl.BlockSpec((1, 1), const),
        ],
        out_specs=[
            pl.BlockSpec((BM, OUT_F), adj_map),
            pl.BlockSpec((BM, IN_F), gn_map),
            pl.BlockSpec((BM, 2), gn_map),
            pl.BlockSpec((BM, 2), gn_map),
        ],
        out_shape=[
            jax.ShapeDtypeStruct((N, OUT_F), f32),
            jax.ShapeDtypeStruct((N, IN_F), f32),
            jax.ShapeDtypeStruct((N, 2), f32),
            jax.ShapeDtypeStruct((N, 2), f32),
        ],
        scratch_shapes=[
            pltpu.VMEM((N, 2 * OUT_F), f32),
            pltpu.VMEM((N, 2 * OUT_F), f32),
            pltpu.VMEM((N, N), jnp.bfloat16),
            pltpu.VMEM((DEPTH, BM, N), f32),
            pltpu.SemaphoreType.DMA((DEPTH,)),
        ],
        compiler_params=pltpu.CompilerParams(
            vmem_limit_bytes=100 * 1024 * 1024,
        ),
        interpret=interpret,
    )(feat, feat_a, weight1, adj, graph_neigh, weight2, disc_w,
      disc_b.reshape(1, 1))

    clustering_loss = jnp.zeros((), f32)
    return (hiden_emb, h, ret, ret_a, clustering_loss)


# speedup vs baseline: 1.0265x; 1.0265x over previous
import functools

import jax
import jax.numpy as jnp
from jax.experimental import pallas as pl
from jax.experimental.pallas import tpu as pltpu

N = 4096
IN_F = 256
OUT_F = 64
BM = 128
NB = N // BM
DEPTH = 4


def _l2norm_sigmoid(x):
    n = jnp.sqrt(jnp.sum(x * x, axis=1, keepdims=True))
    return jax.nn.sigmoid(x / jnp.maximum(n, 1e-12))


def _stream_copy(j, adj_ref, gn_ref, sbuf_ref, sem_ref):
    slot = jax.lax.rem(j, DEPTH)
    adj_cp = pltpu.make_async_copy(
        adj_ref.at[pl.ds(j * BM, BM), :], sbuf_ref.at[slot], sem_ref.at[slot])
    gn_cp = pltpu.make_async_copy(
        gn_ref.at[pl.ds((j - NB) * BM, BM), :], sbuf_ref.at[slot],
        sem_ref.at[slot])
    return adj_cp, gn_cp


def _k_main(feat_ref, feata_ref, w1_ref, adj_ref, gn_ref, w2_ref, dw_ref,
            db_ref, hid_ref, h_ref, ret_ref, reta_ref, m1_ref, zc_ref,
            abf_ref, sbuf_ref, sem_ref):
    i = pl.program_id(0)

    @pl.when(i == 0)
    def _prologue():
        for j in range(DEPTH):
            adj_cp, _ = _stream_copy(jnp.int32(j), adj_ref, gn_ref,
                                     sbuf_ref, sem_ref)
            adj_cp.start()
        w1 = w1_ref[...]
        m1_ref[:, :OUT_F] = jnp.dot(feat_ref[...], w1,
                                    preferred_element_type=jnp.float32)
        m1_ref[:, OUT_F:] = jnp.dot(feata_ref[...], w1,
                                    preferred_element_type=jnp.float32)

    slot = jax.lax.rem(i, DEPTH)
    wait_adj, wait_gn = _stream_copy(i, adj_ref, gn_ref, sbuf_ref, sem_ref)

    @pl.when(i < NB)
    def _wait_b():
        wait_adj.wait()

    @pl.when(i >= NB)
    def _wait_c():
        wait_gn.wait()

    @pl.when(i < NB)
    def _encode():
        ablk = sbuf_ref[slot]
        zb = jnp.dot(ablk, m1_ref[...], preferred_element_type=jnp.float32)
        zc_ref[pl.ds(i * BM, BM), :] = zb
        hid_ref[...] = zb[:, :OUT_F]
        abf_ref[pl.ds(i * BM, BM), :] = ablk.astype(jnp.bfloat16)

    @pl.when(i >= NB)
    def _block():
        r = i - NB
        zcat = zc_ref[...]
        gn = sbuf_ref[slot]

        abf = abf_ref[pl.ds(r * BM, BM), :]
        y = jnp.dot(abf, zcat[:, :OUT_F].astype(jnp.bfloat16),
                    preferred_element_type=jnp.float32)
        h_ref[...] = jnp.dot(y, w2_ref[...], preferred_element_type=jnp.float32)

        emb_full = jnp.maximum(zcat, 0.0)
        vsum = jnp.dot(gn, emb_full, preferred_element_type=jnp.float32)
        rowsum = jnp.sum(gn, axis=1, keepdims=True)
        g = _l2norm_sigmoid(vsum[:, :OUT_F] / rowsum)
        ga = _l2norm_sigmoid(vsum[:, OUT_F:] / rowsum)

        zblk = zc_ref[pl.ds(r * BM, BM), :]
        emb = jnp.maximum(zblk[:, :OUT_F], 0.0)
        emba = jnp.maximum(zblk[:, OUT_F:], 0.0)
        dw = dw_ref[...]
        t = jnp.dot(emb, dw, preferred_element_type=jnp.float32)
        ta = jnp.dot(emba, dw, preferred_element_type=jnp.float32)
        b = db_ref[0, 0]
        sc1 = jnp.sum(t * g, axis=1, keepdims=True) + b
        sc2 = jnp.sum(ta * g, axis=1, keepdims=True) + b
        ret_ref[...] = jnp.concatenate([sc1, sc2], axis=1)
        sc1a = jnp.sum(ta * ga, axis=1, keepdims=True) + b
        sc2a = jnp.sum(t * ga, axis=1, keepdims=True) + b
        reta_ref[...] = jnp.concatenate([sc1a, sc2a], axis=1)

    nxt = i + DEPTH
    next_adj, next_gn = _stream_copy(nxt, adj_ref, gn_ref, sbuf_ref, sem_ref)

    @pl.when(nxt < NB)
    def _issue_b():
        next_adj.start()

    @pl.when(jnp.logical_and(nxt >= NB, nxt < 2 * NB))
    def _issue_c():
        next_gn.start()


@functools.partial(jax.jit, static_argnames=("interpret",))
def kernel(feat, feat_a, adj, graph_neigh, weight1, weight2, disc_w, disc_b,
           interpret=False):
    f32 = jnp.float32
    const = lambda i: (0, 0)
    adj_map = lambda i: (jnp.minimum(i, NB - 1), 0)
    gn_map = lambda i: (jnp.maximum(i - NB, 0), 0)
    hiden_emb, h, ret, ret_a = pl.pallas_call(
        _k_main,
        grid=(2 * NB,),
        in_specs=[
            pl.BlockSpec((N, IN_F), const),
            pl.BlockSpec((N, IN_F), const),
            pl.BlockSpec((IN_F, OUT_F), const),
            pl.BlockSpec(memory_space=pltpu.MemorySpace.HBM),
            pl.BlockSpec(memory_space=pltpu.MemorySpace.HBM),
            pl.BlockSpec((OUT_F, IN_F), const),
            pl.BlockSpec((OUT_F, OUT_F), const),
            pl.BlockSpec((1, 1), const),
        ],
        out_specs=[
            pl.BlockSpec((BM, OUT_F), adj_map),
            pl.BlockSpec((BM, IN_F), gn_map),
            pl.BlockSpec((BM, 2), gn_map),
            pl.BlockSpec((BM, 2), gn_map),
        ],
        out_shape=[
            jax.ShapeDtypeStruct((N, OUT_F), f32),
            jax.ShapeDtypeStruct((N, IN_F), f32),
            jax.ShapeDtypeStruct((N, 2), f32),
            jax.ShapeDtypeStruct((N, 2), f32),
        ],
        scratch_shapes=[
            pltpu.VMEM((N, 2 * OUT_F), f32),
            pltpu.VMEM((N, 2 * OUT_F), f32),
            pltpu.VMEM((N, N), jnp.bfloat16),
            pltpu.VMEM((DEPTH, BM, N), f32),
            pltpu.SemaphoreType.DMA((DEPTH,)),
        ],
        compiler_params=pltpu.CompilerParams(
            vmem_limit_bytes=100 * 1024 * 1024,
        ),
        interpret=interpret,
    )(feat, feat_a, weight1, adj, graph_neigh, weight2, disc_w,
      disc_b.reshape(1, 1))

    clustering_loss = jnp.zeros((), f32)
    return (hiden_emb, h, ret, ret_a, clustering_loss)
